# Initial kernel scaffold; baseline (speedup 1.0000x reference)
import functools

import jax
import jax.numpy as jnp
from jax import lax
from jax.experimental import pallas as pl
from jax.experimental.pallas import tpu as pltpu
from jax.experimental.pallas import tpu_sc as plsc

N = 10000
E = 320000
D = 128

NC = 2
NS = 16
NW = NC * NS
EPT = E // NW
CH = 80
NCHUNK = EPT // CH
RPT = N // NS


def _sc_aggregate(h, src3, dst3, zeros):
    mesh = plsc.VectorSubcoreMesh(core_axis_name="c", subcore_axis_name="s")

    @functools.partial(
        pl.kernel,
        mesh=mesh,
        out_type=jax.ShapeDtypeStruct((NC, N, D), jnp.float32),
        scratch_types=[
            pltpu.VMEM((NCHUNK, CH), jnp.int32),
            pltpu.VMEM((NCHUNK, CH), jnp.int32),
            pltpu.VMEM((CH, D), jnp.float32),
            pltpu.VMEM_SHARED((N, D), jnp.float32),
            pltpu.SemaphoreType.DMA,
        ],
    )
    def agg_kernel(h_hbm, src_hbm, dst_hbm, z_hbm, out_hbm,
                   src_v, dst_v, rows_v, acc_sh, sem):
        cid = lax.axis_index("c")
        sid = lax.axis_index("s")
        wid = sid * NC + cid
        r0 = sid * RPT

        @pl.when(cid == 0)
        def _():
            pltpu.sync_copy(h_hbm.at[pl.ds(r0, RPT)], acc_sh.at[pl.ds(r0, RPT)])

        @pl.when(cid == 1)
        def _():
            pltpu.sync_copy(z_hbm, acc_sh.at[pl.ds(r0, RPT)])

        pltpu.sync_copy(src_hbm.at[wid], src_v)
        pltpu.sync_copy(dst_hbm.at[wid], dst_v)
        plsc.subcore_barrier()

        def body(c, carry):
            pltpu.async_copy(h_hbm.at[src_v.at[c]], rows_v, sem).wait()
            pltpu.sync_copy(rows_v, acc_sh.at[dst_v.at[c]], add=True)
            return carry

        lax.fori_loop(0, NCHUNK, body, 0, unroll=False)

        plsc.subcore_barrier()
        pltpu.sync_copy(acc_sh.at[pl.ds(r0, RPT)],
                        out_hbm.at[cid, pl.ds(r0, RPT)])

    return agg_kernel(h, src3, dst3, zeros)


BLK = 1000


def _mlp_body(p_ref, w1_ref, b1_ref, w2_ref, b2_ref, o_ref):
    z = p_ref[0] + p_ref[1]
    z = jnp.dot(z, w1_ref[...], preferred_element_type=jnp.float32) + b1_ref[...]
    z = jnp.maximum(z, 0.0)
    z = jnp.dot(z, w2_ref[...], preferred_element_type=jnp.float32) + b2_ref[...]
    o_ref[...] = z


def _mlp(p, W1, b1, W2, b2):
    return pl.pallas_call(
        _mlp_body,
        grid=(N // BLK,),
        in_specs=[
            pl.BlockSpec((NC, BLK, D), lambda i: (0, i, 0)),
            pl.BlockSpec((D, D), lambda i: (0, 0)),
            pl.BlockSpec((1, D), lambda i: (0, 0)),
            pl.BlockSpec((D, D), lambda i: (0, 0)),
            pl.BlockSpec((1, D), lambda i: (0, 0)),
        ],
        out_specs=pl.BlockSpec((BLK, D), lambda i: (i, 0)),
        out_shape=jax.ShapeDtypeStruct((N, D), jnp.float32),
    )(p, W1, b1, W2, b2)


def kernel(x, edge_index, W1_0, b1_0, W2_0, b2_0, W1_1, b1_1, W2_1, b2_1):
    src3 = edge_index[0].reshape(NW, NCHUNK, CH)
    dst3 = edge_index[1].reshape(NW, NCHUNK, CH)
    zeros = jnp.zeros((RPT, D), jnp.float32)
    b1_0r = b1_0.reshape(1, D)
    b2_0r = b2_0.reshape(1, D)
    b1_1r = b1_1.reshape(1, D)
    b2_1r = b2_1.reshape(1, D)

    p = _sc_aggregate(x, src3, dst3, zeros)
    h = _mlp(p, W1_0, b1_0r, W2_0, b2_0r)
    p = _sc_aggregate(h, src3, dst3, zeros)
    return _mlp(p, W1_1, b1_1r, W2_1, b2_1r)


# speedup vs baseline: 7.3143x; 7.3143x over previous
import functools

import jax
import jax.numpy as jnp
from jax import lax
from jax.experimental import pallas as pl
from jax.experimental.pallas import tpu as pltpu
from jax.experimental.pallas import tpu_sc as plsc

N = 10000
E = 320000
D = 128

NC = 2
NS = 16
NW = NC * NS
EPT = E // NW
CH = 80
NCHUNK = EPT // CH
R8 = 624
TAIL = N - NS * R8
TAIL_OFF = NS * R8


def _sc_aggregate(h, src3, dst3, zeros):
    mesh = plsc.VectorSubcoreMesh(core_axis_name="c", subcore_axis_name="s")

    @functools.partial(
        pl.kernel,
        mesh=mesh,
        out_type=jax.ShapeDtypeStruct((NC, N, D), jnp.float32),
        scratch_types=[
            pltpu.VMEM((NCHUNK, CH), jnp.int32),
            pltpu.VMEM((NCHUNK, CH), jnp.int32),
            pltpu.VMEM((CH, D), jnp.float32),
            pltpu.VMEM_SHARED((N, D), jnp.float32),
            pltpu.SemaphoreType.DMA,
        ],
    )
    def agg_kernel(h_hbm, src_hbm, dst_hbm, z_hbm, out_hbm,
                   src_v, dst_v, rows_v, acc_sh, sem):
        cid = lax.axis_index("c")
        sid = lax.axis_index("s")
        wid = sid * NC + cid
        r0 = pl.multiple_of(sid * R8, 8)
        last = sid == NS - 1

        @pl.when(cid == 0)
        def _():
            pltpu.sync_copy(h_hbm.at[pl.ds(r0, R8)], acc_sh.at[pl.ds(r0, R8)])

        @pl.when((cid == 0) & last)
        def _():
            pltpu.sync_copy(h_hbm.at[pl.ds(TAIL_OFF, TAIL)],
                            acc_sh.at[pl.ds(TAIL_OFF, TAIL)])

        @pl.when(cid == 1)
        def _():
            pltpu.sync_copy(z_hbm, acc_sh.at[pl.ds(r0, R8)])

        @pl.when((cid == 1) & last)
        def _():
            pltpu.sync_copy(z_hbm.at[pl.ds(0, TAIL)],
                            acc_sh.at[pl.ds(TAIL_OFF, TAIL)])

        pltpu.sync_copy(src_hbm.at[wid], src_v)
        pltpu.sync_copy(dst_hbm.at[wid], dst_v)
        plsc.subcore_barrier()

        def body(c, carry):
            pltpu.async_copy(h_hbm.at[src_v.at[c]], rows_v, sem).wait()
            pltpu.sync_copy(rows_v, acc_sh.at[dst_v.at[c]], add=True)
            return carry

        lax.fori_loop(0, NCHUNK, body, 0, unroll=False)

        plsc.subcore_barrier()
        pltpu.sync_copy(acc_sh.at[pl.ds(r0, R8)],
                        out_hbm.at[cid, pl.ds(r0, R8)])

        @pl.when(last)
        def _():
            pltpu.sync_copy(acc_sh.at[pl.ds(TAIL_OFF, TAIL)],
                            out_hbm.at[cid, pl.ds(TAIL_OFF, TAIL)])

    return agg_kernel(h, src3, dst3, zeros)


BLK = 1000


def _mlp_body(p_ref, w1_ref, b1_ref, w2_ref, b2_ref, o_ref):
    z = p_ref[0] + p_ref[1]
    z = jnp.dot(z, w1_ref[...], preferred_element_type=jnp.float32) + b1_ref[...]
    z = jnp.maximum(z, 0.0)
    z = jnp.dot(z, w2_ref[...], preferred_element_type=jnp.float32) + b2_ref[...]
    o_ref[...] = z


def _mlp(p, W1, b1, W2, b2):
    return pl.pallas_call(
        _mlp_body,
        grid=(N // BLK,),
        in_specs=[
            pl.BlockSpec((NC, BLK, D), lambda i: (0, i, 0)),
            pl.BlockSpec((D, D), lambda i: (0, 0)),
            pl.BlockSpec((1, D), lambda i: (0, 0)),
            pl.BlockSpec((D, D), lambda i: (0, 0)),
            pl.BlockSpec((1, D), lambda i: (0, 0)),
        ],
        out_specs=pl.BlockSpec((BLK, D), lambda i: (i, 0)),
        out_shape=jax.ShapeDtypeStruct((N, D), jnp.float32),
    )(p, W1, b1, W2, b2)


def kernel(x, edge_index, W1_0, b1_0, W2_0, b2_0, W1_1, b1_1, W2_1, b2_1):
    src3 = edge_index[0].reshape(NW, NCHUNK, CH)
    dst3 = edge_index[1].reshape(NW, NCHUNK, CH)
    zeros = jnp.zeros((R8, D), jnp.float32)
    b1_0r = b1_0.reshape(1, D)
    b2_0r = b2_0.reshape(1, D)
    b1_1r = b1_1.reshape(1, D)
    b2_1r = b2_1.reshape(1, D)

    p = _sc_aggregate(x, src3, dst3, zeros)
    h = _mlp(p, W1_0, b1_0r, W2_0, b2_0r)
    p = _sc_aggregate(h, src3, dst3, zeros)
    return _mlp(p, W1_1, b1_1r, W2_1, b2_1r)


# speedup vs baseline: 9.3460x; 1.2778x over previous
import functools

import jax
import jax.numpy as jnp
from jax import lax
from jax.experimental import pallas as pl
from jax.experimental.pallas import tpu as pltpu
from jax.experimental.pallas import tpu_sc as plsc

N = 10000
E = 320000
D = 128

NC = 2
NS = 16
NW = NC * NS
EPT = E // NW
CH = 80
NCHUNK = EPT // CH
RING = 2
R8 = 624
TAIL = N - NS * R8
TAIL_OFF = NS * R8


def _sc_aggregate(h, src3, dst3, zeros):
    mesh = plsc.VectorSubcoreMesh(core_axis_name="c", subcore_axis_name="s")

    @functools.partial(
        pl.kernel,
        mesh=mesh,
        out_type=jax.ShapeDtypeStruct((NC, N, D), jnp.float32),
        scratch_types=[
            pltpu.VMEM((EPT,), jnp.int32),
            pltpu.VMEM((NCHUNK, CH), jnp.int32),
            pltpu.VMEM((RING, CH, D), jnp.float32),
            pltpu.VMEM_SHARED((N, D), jnp.float32),
            pltpu.SemaphoreType.DMA((RING,)),
        ],
    )
    def agg_kernel(h_hbm, src_hbm, dst_hbm, z_hbm, out_hbm,
                   src_v, dst_v, rows_v, acc_sh, gsem):
        cid = lax.axis_index("c")
        sid = lax.axis_index("s")
        wid = sid * NC + cid
        r0 = pl.multiple_of(sid * R8, 8)
        last = sid == NS - 1

        @pl.when(cid == 0)
        def _():
            pltpu.sync_copy(h_hbm.at[pl.ds(r0, R8)], acc_sh.at[pl.ds(r0, R8)])

        @pl.when((cid == 0) & last)
        def _():
            pltpu.sync_copy(h_hbm.at[pl.ds(TAIL_OFF, TAIL)],
                            acc_sh.at[pl.ds(TAIL_OFF, TAIL)])

        @pl.when(cid == 1)
        def _():
            pltpu.sync_copy(z_hbm, acc_sh.at[pl.ds(r0, R8)])

        @pl.when((cid == 1) & last)
        def _():
            pltpu.sync_copy(z_hbm.at[pl.ds(0, TAIL)],
                            acc_sh.at[pl.ds(TAIL_OFF, TAIL)])

        pltpu.sync_copy(src_hbm.at[wid], src_v)
        pltpu.sync_copy(dst_hbm.at[wid], dst_v)
        plsc.subcore_barrier()

        pltpu.async_copy(h_hbm.at[src_v.at[pl.ds(0, CH)]], rows_v.at[0],
                         gsem.at[0])

        def body(c, carry):
            b = lax.rem(c, RING)
            pltpu.make_async_copy(h_hbm.at[src_v.at[pl.ds(c * CH, CH)]],
                                  rows_v.at[b], gsem.at[b]).wait()

            @pl.when(c + 1 < NCHUNK)
            def _():
                bn = lax.rem(c + 1, RING)
                pltpu.async_copy(
                    h_hbm.at[src_v.at[pl.ds((c + 1) * CH, CH)]],
                    rows_v.at[bn], gsem.at[bn])

            pltpu.sync_copy(rows_v.at[b], acc_sh.at[dst_v.at[c]], add=True)
            return carry

        lax.fori_loop(0, NCHUNK, body, 0, unroll=False)

        plsc.subcore_barrier()
        pltpu.sync_copy(acc_sh.at[pl.ds(r0, R8)],
                        out_hbm.at[cid, pl.ds(r0, R8)])

        @pl.when(last)
        def _():
            pltpu.sync_copy(acc_sh.at[pl.ds(TAIL_OFF, TAIL)],
                            out_hbm.at[cid, pl.ds(TAIL_OFF, TAIL)])

    return agg_kernel(h, src3, dst3, zeros)


BLK = 1000


def _mlp_body(p_ref, w1_ref, b1_ref, w2_ref, b2_ref, o_ref):
    z = p_ref[0] + p_ref[1]
    z = jnp.dot(z, w1_ref[...], preferred_element_type=jnp.float32) + b1_ref[...]
    z = jnp.maximum(z, 0.0)
    z = jnp.dot(z, w2_ref[...], preferred_element_type=jnp.float32) + b2_ref[...]
    o_ref[...] = z


def _mlp(p, W1, b1, W2, b2):
    return pl.pallas_call(
        _mlp_body,
        grid=(N // BLK,),
        in_specs=[
            pl.BlockSpec((NC, BLK, D), lambda i: (0, i, 0)),
            pl.BlockSpec((D, D), lambda i: (0, 0)),
            pl.BlockSpec((1, D), lambda i: (0, 0)),
            pl.BlockSpec((D, D), lambda i: (0, 0)),
            pl.BlockSpec((1, D), lambda i: (0, 0)),
        ],
        out_specs=pl.BlockSpec((BLK, D), lambda i: (i, 0)),
        out_shape=jax.ShapeDtypeStruct((N, D), jnp.float32),
    )(p, W1, b1, W2, b2)


def kernel(x, edge_index, W1_0, b1_0, W2_0, b2_0, W1_1, b1_1, W2_1, b2_1):
    src3 = edge_index[0].reshape(NW, EPT)
    dst3 = edge_index[1].reshape(NW, NCHUNK, CH)
    zeros = jnp.zeros((R8, D), jnp.float32)
    b1_0r = b1_0.reshape(1, D)
    b2_0r = b2_0.reshape(1, D)
    b1_1r = b1_1.reshape(1, D)
    b2_1r = b2_1.reshape(1, D)

    p = _sc_aggregate(x, src3, dst3, zeros)
    h = _mlp(p, W1_0, b1_0r, W2_0, b2_0r)
    p = _sc_aggregate(h, src3, dst3, zeros)
    return _mlp(p, W1_1, b1_1r, W2_1, b2_1r)


# speedup vs baseline: 12.0094x; 1.2850x over previous
import functools

import jax
import jax.numpy as jnp
from jax import lax
from jax.experimental import pallas as pl
from jax.experimental.pallas import tpu as pltpu
from jax.experimental.pallas import tpu_sc as plsc

N = 10000
E = 320000
D = 128

NC = 2
NS = 16
NW = NC * NS
EPT = E // NW
CH = 80
NCHUNK = EPT // CH
RING = 2
HALVES = 2
HCH = CH // HALVES
R8 = 624
TAIL = N - NS * R8
TAIL_OFF = NS * R8


def _sc_aggregate(h, src3, dst3, zeros):
    mesh = plsc.VectorSubcoreMesh(core_axis_name="c", subcore_axis_name="s")

    @functools.partial(
        pl.kernel,
        mesh=mesh,
        out_type=jax.ShapeDtypeStruct((NC, N, D), jnp.float32),
        scratch_types=[
            pltpu.VMEM((EPT,), jnp.int32),
            pltpu.VMEM((NCHUNK, CH), jnp.int32),
            pltpu.VMEM((RING, CH, D), jnp.float32),
            pltpu.VMEM_SHARED((N, D), jnp.float32),
            pltpu.SemaphoreType.DMA((2 * RING,)),
        ],
    )
    def agg_kernel(h_hbm, src_hbm, dst_hbm, z_hbm, out_hbm,
                   src_v, dst_v, rows_v, acc_sh, gsem):
        cid = lax.axis_index("c")
        sid = lax.axis_index("s")
        wid = sid * NC + cid
        r0 = pl.multiple_of(sid * R8, 8)
        last = sid == NS - 1

        @pl.when(cid == 0)
        def _():
            pltpu.sync_copy(h_hbm.at[pl.ds(r0, R8)], acc_sh.at[pl.ds(r0, R8)])

        @pl.when((cid == 0) & last)
        def _():
            pltpu.sync_copy(h_hbm.at[pl.ds(TAIL_OFF, TAIL)],
                            acc_sh.at[pl.ds(TAIL_OFF, TAIL)])

        @pl.when(cid == 1)
        def _():
            pltpu.sync_copy(z_hbm, acc_sh.at[pl.ds(r0, R8)])

        @pl.when((cid == 1) & last)
        def _():
            pltpu.sync_copy(z_hbm.at[pl.ds(0, TAIL)],
                            acc_sh.at[pl.ds(TAIL_OFF, TAIL)])

        pltpu.sync_copy(src_hbm.at[wid], src_v)
        pltpu.sync_copy(dst_hbm.at[wid], dst_v)
        plsc.subcore_barrier()

        def issue_halves(cc):
            bn = lax.rem(cc, RING)
            for hh in range(HALVES):
                pltpu.async_copy(
                    h_hbm.at[src_v.at[pl.ds(cc * CH + hh * HCH, HCH)]],
                    rows_v.at[bn, pl.ds(hh * HCH, HCH)],
                    gsem.at[HALVES * bn + hh])

        def wait_halves(cc):
            bn = lax.rem(cc, RING)
            for hh in range(HALVES):
                pltpu.make_async_copy(
                    h_hbm.at[src_v.at[pl.ds(cc * CH + hh * HCH, HCH)]],
                    rows_v.at[bn, pl.ds(hh * HCH, HCH)],
                    gsem.at[HALVES * bn + hh]).wait()

        issue_halves(0)

        def body(c, carry):
            @pl.when(c + 1 < NCHUNK)
            def _():
                issue_halves(c + 1)

            wait_halves(c)
            pltpu.sync_copy(rows_v.at[lax.rem(c, RING)],
                            acc_sh.at[dst_v.at[c]], add=True)
            return carry

        lax.fori_loop(0, NCHUNK, body, 0, unroll=False)

        plsc.subcore_barrier()
        pltpu.sync_copy(acc_sh.at[pl.ds(r0, R8)],
                        out_hbm.at[cid, pl.ds(r0, R8)])

        @pl.when(last)
        def _():
            pltpu.sync_copy(acc_sh.at[pl.ds(TAIL_OFF, TAIL)],
                            out_hbm.at[cid, pl.ds(TAIL_OFF, TAIL)])

    return agg_kernel(h, src3, dst3, zeros)


BLK = 1000


def _mlp_body(p_ref, w1_ref, b1_ref, w2_ref, b2_ref, o_ref):
    z = p_ref[0] + p_ref[1]
    z = jnp.dot(z, w1_ref[...], preferred_element_type=jnp.float32) + b1_ref[...]
    z = jnp.maximum(z, 0.0)
    z = jnp.dot(z, w2_ref[...], preferred_element_type=jnp.float32) + b2_ref[...]
    o_ref[...] = z


def _mlp(p, W1, b1, W2, b2):
    return pl.pallas_call(
        _mlp_body,
        grid=(N // BLK,),
        in_specs=[
            pl.BlockSpec((NC, BLK, D), lambda i: (0, i, 0)),
            pl.BlockSpec((D, D), lambda i: (0, 0)),
            pl.BlockSpec((1, D), lambda i: (0, 0)),
            pl.BlockSpec((D, D), lambda i: (0, 0)),
            pl.BlockSpec((1, D), lambda i: (0, 0)),
        ],
        out_specs=pl.BlockSpec((BLK, D), lambda i: (i, 0)),
        out_shape=jax.ShapeDtypeStruct((N, D), jnp.float32),
    )(p, W1, b1, W2, b2)


def kernel(x, edge_index, W1_0, b1_0, W2_0, b2_0, W1_1, b1_1, W2_1, b2_1):
    src3 = edge_index[0].reshape(NW, EPT)
    dst3 = edge_index[1].reshape(NW, NCHUNK, CH)
    zeros = jnp.zeros((R8, D), jnp.float32)
    b1_0r = b1_0.reshape(1, D)
    b2_0r = b2_0.reshape(1, D)
    b1_1r = b1_1.reshape(1, D)
    b2_1r = b2_1.reshape(1, D)

    p = _sc_aggregate(x, src3, dst3, zeros)
    h = _mlp(p, W1_0, b1_0r, W2_0, b2_0r)
    p = _sc_aggregate(h, src3, dst3, zeros)
    return _mlp(p, W1_1, b1_1r, W2_1, b2_1r)


# speedup vs baseline: 13.5970x; 1.1322x over previous
import functools

import jax
import jax.numpy as jnp
from jax import lax
from jax.experimental import pallas as pl
from jax.experimental.pallas import tpu as pltpu
from jax.experimental.pallas import tpu_sc as plsc

N = 10000
E = 320000
D = 128

NC = 2
NS = 16
NW = NC * NS
EPT = E // NW
CH = 80
NCHUNK = EPT // CH
RING = 3
HALVES = 2
HCH = CH // HALVES
R8 = 624
TAIL = N - NS * R8
TAIL_OFF = NS * R8


def _sc_aggregate(h, src3, dst3, zeros):
    mesh = plsc.VectorSubcoreMesh(core_axis_name="c", subcore_axis_name="s")

    @functools.partial(
        pl.kernel,
        mesh=mesh,
        out_type=jax.ShapeDtypeStruct((NC, N, D), jnp.float32),
        scratch_types=[
            pltpu.VMEM((EPT,), jnp.int32),
            pltpu.VMEM((RING, 1, CH), jnp.int32),
            pltpu.VMEM((RING, CH, D), jnp.float32),
            pltpu.VMEM_SHARED((N, D), jnp.float32),
            pltpu.SemaphoreType.DMA((HALVES * RING,)),
            pltpu.SemaphoreType.DMA((RING,)),
        ],
    )
    def agg_kernel(h_hbm, src_hbm, dst_hbm, z_hbm, out_hbm,
                   src_v, dst_v, rows_v, acc_sh, gsem, dsem):
        cid = lax.axis_index("c")
        sid = lax.axis_index("s")
        wid = sid * NC + cid
        r0 = pl.multiple_of(sid * R8, 8)
        last = sid == NS - 1

        @pl.when(cid == 0)
        def _():
            pltpu.sync_copy(h_hbm.at[pl.ds(r0, R8)], acc_sh.at[pl.ds(r0, R8)])

        @pl.when((cid == 0) & last)
        def _():
            pltpu.sync_copy(h_hbm.at[pl.ds(TAIL_OFF, TAIL)],
                            acc_sh.at[pl.ds(TAIL_OFF, TAIL)])

        @pl.when(cid == 1)
        def _():
            pltpu.sync_copy(z_hbm, acc_sh.at[pl.ds(r0, R8)])

        @pl.when((cid == 1) & last)
        def _():
            pltpu.sync_copy(z_hbm.at[pl.ds(0, TAIL)],
                            acc_sh.at[pl.ds(TAIL_OFF, TAIL)])

        pltpu.sync_copy(src_hbm.at[wid], src_v)
        plsc.subcore_barrier()

        def issue_chunk(cc):
            bn = lax.rem(cc, RING)
            pltpu.async_copy(dst_hbm.at[wid, cc], dst_v.at[bn],
                             dsem.at[bn])
            for hh in range(HALVES):
                pltpu.async_copy(
                    h_hbm.at[src_v.at[pl.ds(cc * CH + hh * HCH, HCH)]],
                    rows_v.at[bn, pl.ds(hh * HCH, HCH)],
                    gsem.at[HALVES * bn + hh])

        def wait_chunk(cc):
            bn = lax.rem(cc, RING)
            pltpu.make_async_copy(dst_hbm.at[wid, cc], dst_v.at[bn],
                                  dsem.at[bn]).wait()
            for hh in range(HALVES):
                pltpu.make_async_copy(
                    h_hbm.at[src_v.at[pl.ds(cc * CH + hh * HCH, HCH)]],
                    rows_v.at[bn, pl.ds(hh * HCH, HCH)],
                    gsem.at[HALVES * bn + hh]).wait()

        for c0 in range(RING):
            issue_chunk(c0)

        def body(c, carry):
            bn = lax.rem(c, RING)
            wait_chunk(c)
            pltpu.sync_copy(rows_v.at[bn], acc_sh.at[dst_v.at[bn, 0]],
                            add=True)

            @pl.when(c + RING < NCHUNK)
            def _():
                issue_chunk(c + RING)

            return carry

        lax.fori_loop(0, NCHUNK, body, 0, unroll=False)

        plsc.subcore_barrier()
        pltpu.sync_copy(acc_sh.at[pl.ds(r0, R8)],
                        out_hbm.at[cid, pl.ds(r0, R8)])

        @pl.when(last)
        def _():
            pltpu.sync_copy(acc_sh.at[pl.ds(TAIL_OFF, TAIL)],
                            out_hbm.at[cid, pl.ds(TAIL_OFF, TAIL)])

    return agg_kernel(h, src3, dst3, zeros)


BLK = 1000


def _mlp_body(p_ref, w1_ref, b1_ref, w2_ref, b2_ref, o_ref):
    z = p_ref[0] + p_ref[1]
    z = jnp.dot(z, w1_ref[...], preferred_element_type=jnp.float32) + b1_ref[...]
    z = jnp.maximum(z, 0.0)
    z = jnp.dot(z, w2_ref[...], preferred_element_type=jnp.float32) + b2_ref[...]
    o_ref[...] = z


def _mlp(p, W1, b1, W2, b2):
    return pl.pallas_call(
        _mlp_body,
        grid=(N // BLK,),
        in_specs=[
            pl.BlockSpec((NC, BLK, D), lambda i: (0, i, 0)),
            pl.BlockSpec((D, D), lambda i: (0, 0)),
            pl.BlockSpec((1, D), lambda i: (0, 0)),
            pl.BlockSpec((D, D), lambda i: (0, 0)),
            pl.BlockSpec((1, D), lambda i: (0, 0)),
        ],
        out_specs=pl.BlockSpec((BLK, D), lambda i: (i, 0)),
        out_shape=jax.ShapeDtypeStruct((N, D), jnp.float32),
    )(p, W1, b1, W2, b2)


def kernel(x, edge_index, W1_0, b1_0, W2_0, b2_0, W1_1, b1_1, W2_1, b2_1):
    src3 = edge_index[0].reshape(NW, EPT)
    dst3 = edge_index[1].reshape(NW, NCHUNK, 1, CH)
    zeros = jnp.zeros((R8, D), jnp.float32)
    b1_0r = b1_0.reshape(1, D)
    b2_0r = b2_0.reshape(1, D)
    b1_1r = b1_1.reshape(1, D)
    b2_1r = b2_1.reshape(1, D)

    p = _sc_aggregate(x, src3, dst3, zeros)
    h = _mlp(p, W1_0, b1_0r, W2_0, b2_0r)
    p = _sc_aggregate(h, src3, dst3, zeros)
    return _mlp(p, W1_1, b1_1r, W2_1, b2_1r)


# speedup vs baseline: 13.5984x; 1.0001x over previous
import functools

import jax
import jax.numpy as jnp
from jax import lax
from jax.experimental import pallas as pl
from jax.experimental.pallas import tpu as pltpu
from jax.experimental.pallas import tpu_sc as plsc

N = 10000
E = 320000
D = 128

NC = 2
NS = 16
NW = NC * NS
EPT = E // NW
CH = 80
NCHUNK = EPT // CH
RING = 3
HALVES = 2
HCH = CH // HALVES
R8 = 624
TAIL = N - NS * R8
TAIL_OFF = NS * R8


def _sc_aggregate(h, src3, dst3, zeros):
    mesh = plsc.VectorSubcoreMesh(core_axis_name="c", subcore_axis_name="s")

    @functools.partial(
        pl.kernel,
        mesh=mesh,
        out_type=jax.ShapeDtypeStruct((NC, N, D), jnp.float32),
        scratch_types=[
            pltpu.VMEM((EPT,), jnp.int32),
            pltpu.VMEM((RING, 1, CH), jnp.int32),
            pltpu.VMEM((RING, CH, D), jnp.float32),
            pltpu.VMEM_SHARED((N, D), jnp.float32),
            pltpu.SemaphoreType.DMA((HALVES * RING,)),
            pltpu.SemaphoreType.DMA((RING,)),
            pltpu.SemaphoreType.DMA((RING,)),
        ],
    )
    def agg_kernel(h_hbm, src_hbm, dst_hbm, z_hbm, out_hbm,
                   src_v, dst_v, rows_v, acc_sh, gsem, dsem, ssem):
        cid = lax.axis_index("c")
        sid = lax.axis_index("s")
        wid = sid * NC + cid
        r0 = pl.multiple_of(sid * R8, 8)
        last = sid == NS - 1

        @pl.when(cid == 0)
        def _():
            pltpu.sync_copy(h_hbm.at[pl.ds(r0, R8)], acc_sh.at[pl.ds(r0, R8)])

        @pl.when((cid == 0) & last)
        def _():
            pltpu.sync_copy(h_hbm.at[pl.ds(TAIL_OFF, TAIL)],
                            acc_sh.at[pl.ds(TAIL_OFF, TAIL)])

        @pl.when(cid == 1)
        def _():
            pltpu.sync_copy(z_hbm, acc_sh.at[pl.ds(r0, R8)])

        @pl.when((cid == 1) & last)
        def _():
            pltpu.sync_copy(z_hbm.at[pl.ds(0, TAIL)],
                            acc_sh.at[pl.ds(TAIL_OFF, TAIL)])

        pltpu.sync_copy(src_hbm.at[wid], src_v)
        plsc.subcore_barrier()

        def issue_chunk(cc):
            bn = lax.rem(cc, RING)
            pltpu.async_copy(dst_hbm.at[wid, cc], dst_v.at[bn],
                             dsem.at[bn])
            for hh in range(HALVES):
                pltpu.async_copy(
                    h_hbm.at[src_v.at[pl.ds(cc * CH + hh * HCH, HCH)]],
                    rows_v.at[bn, pl.ds(hh * HCH, HCH)],
                    gsem.at[HALVES * bn + hh])

        def wait_chunk(cc):
            bn = lax.rem(cc, RING)
            pltpu.make_async_copy(dst_hbm.at[wid, cc], dst_v.at[bn],
                                  dsem.at[bn]).wait()
            for hh in range(HALVES):
                pltpu.make_async_copy(
                    h_hbm.at[src_v.at[pl.ds(cc * CH + hh * HCH, HCH)]],
                    rows_v.at[bn, pl.ds(hh * HCH, HCH)],
                    gsem.at[HALVES * bn + hh]).wait()

        def drain_scatter(cc):
            bp = lax.rem(cc, RING)
            pltpu.make_async_copy(rows_v.at[bp], acc_sh.at[dst_v.at[bp, 0]],
                                  ssem.at[bp]).wait()

        for c0 in range(RING - 1):
            issue_chunk(c0)

        def body(c, carry):
            bn = lax.rem(c, RING)

            @pl.when(c >= 1)
            def _():
                drain_scatter(c - 1)

            @pl.when(c + RING - 1 < NCHUNK)
            def _():
                issue_chunk(c + RING - 1)

            wait_chunk(c)
            pltpu.async_copy(rows_v.at[bn], acc_sh.at[dst_v.at[bn, 0]],
                             ssem.at[bn], add=True)
            return carry

        lax.fori_loop(0, NCHUNK, body, 0, unroll=False)
        drain_scatter(NCHUNK - 1)

        plsc.subcore_barrier()
        pltpu.sync_copy(acc_sh.at[pl.ds(r0, R8)],
                        out_hbm.at[cid, pl.ds(r0, R8)])

        @pl.when(last)
        def _():
            pltpu.sync_copy(acc_sh.at[pl.ds(TAIL_OFF, TAIL)],
                            out_hbm.at[cid, pl.ds(TAIL_OFF, TAIL)])

    return agg_kernel(h, src3, dst3, zeros)


BLK = 1000


def _mlp_body(p_ref, w1_ref, b1_ref, w2_ref, b2_ref, o_ref):
    z = p_ref[0] + p_ref[1]
    z = jnp.dot(z, w1_ref[...], preferred_element_type=jnp.float32) + b1_ref[...]
    z = jnp.maximum(z, 0.0)
    z = jnp.dot(z, w2_ref[...], preferred_element_type=jnp.float32) + b2_ref[...]
    o_ref[...] = z


def _mlp(p, W1, b1, W2, b2):
    return pl.pallas_call(
        _mlp_body,
        grid=(N // BLK,),
        in_specs=[
            pl.BlockSpec((NC, BLK, D), lambda i: (0, i, 0)),
            pl.BlockSpec((D, D), lambda i: (0, 0)),
            pl.BlockSpec((1, D), lambda i: (0, 0)),
            pl.BlockSpec((D, D), lambda i: (0, 0)),
            pl.BlockSpec((1, D), lambda i: (0, 0)),
        ],
        out_specs=pl.BlockSpec((BLK, D), lambda i: (i, 0)),
        out_shape=jax.ShapeDtypeStruct((N, D), jnp.float32),
    )(p, W1, b1, W2, b2)


def kernel(x, edge_index, W1_0, b1_0, W2_0, b2_0, W1_1, b1_1, W2_1, b2_1):
    src3 = edge_index[0].reshape(NW, EPT)
    dst3 = edge_index[1].reshape(NW, NCHUNK, 1, CH)
    zeros = jnp.zeros((R8, D), jnp.float32)
    b1_0r = b1_0.reshape(1, D)
    b2_0r = b2_0.reshape(1, D)
    b1_1r = b1_1.reshape(1, D)
    b2_1r = b2_1.reshape(1, D)

    p = _sc_aggregate(x, src3, dst3, zeros)
    h = _mlp(p, W1_0, b1_0r, W2_0, b2_0r)
    p = _sc_aggregate(h, src3, dst3, zeros)
    return _mlp(p, W1_1, b1_1r, W2_1, b2_1r)


# speedup vs baseline: 14.3578x; 1.0558x over previous
import functools

import jax
import jax.numpy as jnp
from jax import lax
from jax.experimental import pallas as pl
from jax.experimental.pallas import tpu as pltpu
from jax.experimental.pallas import tpu_sc as plsc

N = 10000
E = 320000
D = 128

NC = 2
NS = 16
NW = NC * NS
EPT = E // NW
CH = 80
NCHUNK = EPT // CH
RING = 3
HALVES = 2
HCH = CH // HALVES
R8 = 624
TAIL = N - NS * R8
TAIL_OFF = NS * R8


def _sc_aggregate(h, src1, dst1):
    mesh = plsc.VectorSubcoreMesh(core_axis_name="c", subcore_axis_name="s")

    @functools.partial(
        pl.kernel,
        mesh=mesh,
        out_type=jax.ShapeDtypeStruct((NC, N, D), jnp.float32),
        scratch_types=[
            pltpu.VMEM((EPT,), jnp.int32),
            pltpu.VMEM((RING, CH), jnp.int32),
            pltpu.VMEM((RING, CH, D), jnp.float32),
            pltpu.VMEM_SHARED((N, D), jnp.float32),
            pltpu.SemaphoreType.DMA((HALVES * RING,)),
            pltpu.SemaphoreType.DMA((RING,)),
            pltpu.SemaphoreType.DMA((RING,)),
        ],
    )
    def agg_kernel(h_hbm, src_hbm, dst_hbm, out_hbm,
                   src_v, dst_v, rows_v, acc_sh, gsem, dsem, ssem):
        cid = lax.axis_index("c")
        sid = lax.axis_index("s")
        wid = sid * NC + cid
        r0 = pl.multiple_of(sid * R8, 8)
        last = sid == NS - 1

        @pl.when(cid == 0)
        def _():
            pltpu.sync_copy(h_hbm.at[pl.ds(r0, R8)], acc_sh.at[pl.ds(r0, R8)])

        @pl.when((cid == 0) & last)
        def _():
            pltpu.sync_copy(h_hbm.at[pl.ds(TAIL_OFF, TAIL)],
                            acc_sh.at[pl.ds(TAIL_OFF, TAIL)])

        @pl.when(cid == 1)
        def _():
            def zrow(r, carry):
                for m in range(D // 16):
                    rows_v[0, r, pl.ds(m * 16, 16)] = jnp.zeros(
                        (16,), jnp.float32)
                return carry

            lax.fori_loop(0, CH, zrow, 0, unroll=False)
            for k in range(7):
                pltpu.sync_copy(rows_v.at[0],
                                acc_sh.at[pl.ds(r0 + k * CH, CH)])
            pltpu.sync_copy(rows_v.at[0, pl.ds(0, R8 - 7 * CH)],
                            acc_sh.at[pl.ds(r0 + 7 * CH, R8 - 7 * CH)])

        @pl.when((cid == 1) & last)
        def _():
            pltpu.sync_copy(rows_v.at[0, pl.ds(0, TAIL)],
                            acc_sh.at[pl.ds(TAIL_OFF, TAIL)])

        pltpu.sync_copy(src_hbm.at[pl.ds(wid * EPT, EPT)], src_v)
        plsc.subcore_barrier()

        def issue_chunk(cc):
            bn = lax.rem(cc, RING)
            pltpu.async_copy(dst_hbm.at[pl.ds(wid * EPT + cc * CH, CH)],
                             dst_v.at[bn], dsem.at[bn])
            for hh in range(HALVES):
                pltpu.async_copy(
                    h_hbm.at[src_v.at[pl.ds(cc * CH + hh * HCH, HCH)]],
                    rows_v.at[bn, pl.ds(hh * HCH, HCH)],
                    gsem.at[HALVES * bn + hh])

        def wait_chunk(cc):
            bn = lax.rem(cc, RING)
            pltpu.make_async_copy(dst_hbm.at[pl.ds(wid * EPT + cc * CH, CH)],
                                  dst_v.at[bn], dsem.at[bn]).wait()
            for hh in range(HALVES):
                pltpu.make_async_copy(
                    h_hbm.at[src_v.at[pl.ds(cc * CH + hh * HCH, HCH)]],
                    rows_v.at[bn, pl.ds(hh * HCH, HCH)],
                    gsem.at[HALVES * bn + hh]).wait()

        def drain_scatter(cc):
            bp = lax.rem(cc, RING)
            pltpu.make_async_copy(rows_v.at[bp], acc_sh.at[dst_v.at[bp]],
                                  ssem.at[bp]).wait()

        for c0 in range(RING - 1):
            issue_chunk(c0)

        def body(c, carry):
            bn = lax.rem(c, RING)

            @pl.when(c >= 1)
            def _():
                drain_scatter(c - 1)

            @pl.when(c + RING - 1 < NCHUNK)
            def _():
                issue_chunk(c + RING - 1)

            wait_chunk(c)
            pltpu.async_copy(rows_v.at[bn], acc_sh.at[dst_v.at[bn]],
                             ssem.at[bn], add=True)
            return carry

        lax.fori_loop(0, NCHUNK, body, 0, unroll=False)
        drain_scatter(NCHUNK - 1)

        plsc.subcore_barrier()
        pltpu.sync_copy(acc_sh.at[pl.ds(r0, R8)],
                        out_hbm.at[cid, pl.ds(r0, R8)])

        @pl.when(last)
        def _():
            pltpu.sync_copy(acc_sh.at[pl.ds(TAIL_OFF, TAIL)],
                            out_hbm.at[cid, pl.ds(TAIL_OFF, TAIL)])

    return agg_kernel(h, src1, dst1)


BLK = 2000


def _mlp_body(p_ref, w1_ref, b1_ref, w2_ref, b2_ref, o_ref):
    z = p_ref[0] + p_ref[1]
    z = jnp.dot(z, w1_ref[...], preferred_element_type=jnp.float32) + b1_ref[...]
    z = jnp.maximum(z, 0.0)
    z = jnp.dot(z, w2_ref[...], preferred_element_type=jnp.float32) + b2_ref[...]
    o_ref[...] = z


def _mlp(p, W1, b1, W2, b2):
    return pl.pallas_call(
        _mlp_body,
        grid=(N // BLK,),
        in_specs=[
            pl.BlockSpec((NC, BLK, D), lambda i: (0, i, 0)),
            pl.BlockSpec((D, D), lambda i: (0, 0)),
            pl.BlockSpec((1, D), lambda i: (0, 0)),
            pl.BlockSpec((D, D), lambda i: (0, 0)),
            pl.BlockSpec((1, D), lambda i: (0, 0)),
        ],
        out_specs=pl.BlockSpec((BLK, D), lambda i: (i, 0)),
        out_shape=jax.ShapeDtypeStruct((N, D), jnp.float32),
    )(p, W1, b1, W2, b2)


def kernel(x, edge_index, W1_0, b1_0, W2_0, b2_0, W1_1, b1_1, W2_1, b2_1):
    src1 = edge_index[0]
    dst1 = edge_index[1]
    b1_0r = b1_0.reshape(1, D)
    b2_0r = b2_0.reshape(1, D)
    b1_1r = b1_1.reshape(1, D)
    b2_1r = b2_1.reshape(1, D)

    p = _sc_aggregate(x, src1, dst1)
    h = _mlp(p, W1_0, b1_0r, W2_0, b2_0r)
    p = _sc_aggregate(h, src1, dst1)
    return _mlp(p, W1_1, b1_1r, W2_1, b2_1r)


# speedup vs baseline: 14.7019x; 1.0240x over previous
import functools

import jax
import jax.numpy as jnp
from jax import lax
from jax.experimental import pallas as pl
from jax.experimental.pallas import tpu as pltpu
from jax.experimental.pallas import tpu_sc as plsc

N = 10000
E = 320000
D = 128

NC = 2
NS = 16
NW = NC * NS
EPT = E // NW
CH = 40
NCHUNK = EPT // CH
RING = 6
LOOK = RING - 2
HALVES = 1
HCH = CH // HALVES
R8 = 624
TAIL = N - NS * R8
TAIL_OFF = NS * R8


def _sc_aggregate(h, src1, dst1):
    mesh = plsc.VectorSubcoreMesh(core_axis_name="c", subcore_axis_name="s")

    @functools.partial(
        pl.kernel,
        mesh=mesh,
        out_type=jax.ShapeDtypeStruct((NC, N, D), jnp.float32),
        scratch_types=[
            pltpu.VMEM((EPT,), jnp.int32),
            pltpu.VMEM((RING, CH), jnp.int32),
            pltpu.VMEM((RING, CH, D), jnp.float32),
            pltpu.VMEM_SHARED((N, D), jnp.float32),
            pltpu.SemaphoreType.DMA((HALVES * RING,)),
            pltpu.SemaphoreType.DMA((RING,)),
            pltpu.SemaphoreType.DMA((RING,)),
        ],
    )
    def agg_kernel(h_hbm, src_hbm, dst_hbm, out_hbm,
                   src_v, dst_v, rows_v, acc_sh, gsem, dsem, ssem):
        cid = lax.axis_index("c")
        sid = lax.axis_index("s")
        wid = sid * NC + cid
        r0 = pl.multiple_of(sid * R8, 8)
        last = sid == NS - 1

        @pl.when(cid == 0)
        def _():
            pltpu.sync_copy(h_hbm.at[pl.ds(r0, R8)], acc_sh.at[pl.ds(r0, R8)])

        @pl.when((cid == 0) & last)
        def _():
            pltpu.sync_copy(h_hbm.at[pl.ds(TAIL_OFF, TAIL)],
                            acc_sh.at[pl.ds(TAIL_OFF, TAIL)])

        @pl.when(cid == 1)
        def _():
            def zrow(r, carry):
                for m in range(D // 16):
                    rows_v[0, r, pl.ds(m * 16, 16)] = jnp.zeros(
                        (16,), jnp.float32)
                return carry

            lax.fori_loop(0, CH, zrow, 0, unroll=False)
            for k in range(R8 // CH):
                pltpu.sync_copy(rows_v.at[0],
                                acc_sh.at[pl.ds(r0 + k * CH, CH)])
            if R8 % CH:
                pltpu.sync_copy(rows_v.at[0, pl.ds(0, R8 % CH)],
                                acc_sh.at[pl.ds(r0 + (R8 // CH) * CH,
                                                R8 % CH)])

        @pl.when((cid == 1) & last)
        def _():
            pltpu.sync_copy(rows_v.at[0, pl.ds(0, TAIL)],
                            acc_sh.at[pl.ds(TAIL_OFF, TAIL)])

        pltpu.sync_copy(src_hbm.at[pl.ds(wid * EPT, EPT)], src_v)
        plsc.subcore_barrier()

        def issue_chunk(cc):
            bn = lax.rem(cc, RING)
            pltpu.async_copy(dst_hbm.at[pl.ds(wid * EPT + cc * CH, CH)],
                             dst_v.at[bn], dsem.at[bn])
            for hh in range(HALVES):
                pltpu.async_copy(
                    h_hbm.at[src_v.at[pl.ds(cc * CH + hh * HCH, HCH)]],
                    rows_v.at[bn, pl.ds(hh * HCH, HCH)],
                    gsem.at[HALVES * bn + hh])

        def wait_chunk(cc):
            bn = lax.rem(cc, RING)
            pltpu.make_async_copy(dst_hbm.at[pl.ds(wid * EPT + cc * CH, CH)],
                                  dst_v.at[bn], dsem.at[bn]).wait()
            for hh in range(HALVES):
                pltpu.make_async_copy(
                    h_hbm.at[src_v.at[pl.ds(cc * CH + hh * HCH, HCH)]],
                    rows_v.at[bn, pl.ds(hh * HCH, HCH)],
                    gsem.at[HALVES * bn + hh]).wait()

        def drain_scatter(cc):
            bp = lax.rem(cc, RING)
            pltpu.make_async_copy(rows_v.at[bp], acc_sh.at[dst_v.at[bp]],
                                  ssem.at[bp]).wait()

        for c0 in range(LOOK):
            issue_chunk(c0)

        def body(c, carry):
            bn = lax.rem(c, RING)

            @pl.when(c >= RING - LOOK)
            def _():
                drain_scatter(c - (RING - LOOK))

            @pl.when(c + LOOK < NCHUNK)
            def _():
                issue_chunk(c + LOOK)

            wait_chunk(c)
            pltpu.async_copy(rows_v.at[bn], acc_sh.at[dst_v.at[bn]],
                             ssem.at[bn], add=True)
            return carry

        lax.fori_loop(0, NCHUNK, body, 0, unroll=False)
        for cc in range(NCHUNK - (RING - LOOK), NCHUNK):
            bp = cc % RING
            pltpu.make_async_copy(rows_v.at[bp], acc_sh.at[dst_v.at[bp]],
                                  ssem.at[bp]).wait()

        plsc.subcore_barrier()
        pltpu.sync_copy(acc_sh.at[pl.ds(r0, R8)],
                        out_hbm.at[cid, pl.ds(r0, R8)])

        @pl.when(last)
        def _():
            pltpu.sync_copy(acc_sh.at[pl.ds(TAIL_OFF, TAIL)],
                            out_hbm.at[cid, pl.ds(TAIL_OFF, TAIL)])

    return agg_kernel(h, src1, dst1)


BLK = 2000


def _mlp_body(p_ref, w1_ref, b1_ref, w2_ref, b2_ref, o_ref):
    z = p_ref[0] + p_ref[1]
    z = jnp.dot(z, w1_ref[...], preferred_element_type=jnp.float32) + b1_ref[...]
    z = jnp.maximum(z, 0.0)
    z = jnp.dot(z, w2_ref[...], preferred_element_type=jnp.float32) + b2_ref[...]
    o_ref[...] = z


def _mlp(p, W1, b1, W2, b2):
    return pl.pallas_call(
        _mlp_body,
        grid=(N // BLK,),
        in_specs=[
            pl.BlockSpec((NC, BLK, D), lambda i: (0, i, 0)),
            pl.BlockSpec((D, D), lambda i: (0, 0)),
            pl.BlockSpec((1, D), lambda i: (0, 0)),
            pl.BlockSpec((D, D), lambda i: (0, 0)),
            pl.BlockSpec((1, D), lambda i: (0, 0)),
        ],
        out_specs=pl.BlockSpec((BLK, D), lambda i: (i, 0)),
        out_shape=jax.ShapeDtypeStruct((N, D), jnp.float32),
    )(p, W1, b1, W2, b2)


def kernel(x, edge_index, W1_0, b1_0, W2_0, b2_0, W1_1, b1_1, W2_1, b2_1):
    src1 = edge_index[0]
    dst1 = edge_index[1]
    b1_0r = b1_0.reshape(1, D)
    b2_0r = b2_0.reshape(1, D)
    b1_1r = b1_1.reshape(1, D)
    b2_1r = b2_1.reshape(1, D)

    p = _sc_aggregate(x, src1, dst1)
    h = _mlp(p, W1_0, b1_0r, W2_0, b2_0r)
    p = _sc_aggregate(h, src1, dst1)
    return _mlp(p, W1_1, b1_1r, W2_1, b2_1r)


# speedup vs baseline: 14.8201x; 1.0080x over previous
import functools

import jax
import jax.numpy as jnp
from jax import lax
from jax.experimental import pallas as pl
from jax.experimental.pallas import tpu as pltpu
from jax.experimental.pallas import tpu_sc as plsc

N = 10000
E = 320000
D = 128

NC = 2
NS = 16
NW = NC * NS
EPT = E // NW
CH = 40
NCHUNK = EPT // CH
RING = 7
LOOK = RING - 2
HALVES = 1
HCH = CH // HALVES
R8 = 624
TAIL = N - NS * R8
TAIL_OFF = NS * R8


def _sc_aggregate(h, src1, dst1):
    mesh = plsc.VectorSubcoreMesh(core_axis_name="c", subcore_axis_name="s")

    @functools.partial(
        pl.kernel,
        mesh=mesh,
        out_type=jax.ShapeDtypeStruct((NC, N, D), jnp.float32),
        scratch_types=[
            pltpu.VMEM((EPT,), jnp.int32),
            pltpu.VMEM((RING, CH), jnp.int32),
            pltpu.VMEM((RING, CH, D), jnp.float32),
            pltpu.VMEM_SHARED((N, D), jnp.float32),
            pltpu.SemaphoreType.DMA((HALVES * RING,)),
            pltpu.SemaphoreType.DMA((RING,)),
            pltpu.SemaphoreType.DMA((RING,)),
        ],
    )
    def agg_kernel(h_hbm, src_hbm, dst_hbm, out_hbm,
                   src_v, dst_v, rows_v, acc_sh, gsem, dsem, ssem):
        cid = lax.axis_index("c")
        sid = lax.axis_index("s")
        wid = sid * NC + cid
        r0 = pl.multiple_of(sid * R8, 8)
        last = sid == NS - 1

        @pl.when(cid == 0)
        def _():
            pltpu.sync_copy(h_hbm.at[pl.ds(r0, R8)], acc_sh.at[pl.ds(r0, R8)])

        @pl.when((cid == 0) & last)
        def _():
            pltpu.sync_copy(h_hbm.at[pl.ds(TAIL_OFF, TAIL)],
                            acc_sh.at[pl.ds(TAIL_OFF, TAIL)])

        @pl.when(cid == 1)
        def _():
            def zrow(r, carry):
                for m in range(D // 16):
                    rows_v[0, r, pl.ds(m * 16, 16)] = jnp.zeros(
                        (16,), jnp.float32)
                return carry

            lax.fori_loop(0, CH, zrow, 0, unroll=False)
            for k in range(R8 // CH):
                pltpu.sync_copy(rows_v.at[0],
                                acc_sh.at[pl.ds(r0 + k * CH, CH)])
            if R8 % CH:
                pltpu.sync_copy(rows_v.at[0, pl.ds(0, R8 % CH)],
                                acc_sh.at[pl.ds(r0 + (R8 // CH) * CH,
                                                R8 % CH)])

        @pl.when((cid == 1) & last)
        def _():
            pltpu.sync_copy(rows_v.at[0, pl.ds(0, TAIL)],
                            acc_sh.at[pl.ds(TAIL_OFF, TAIL)])

        pltpu.sync_copy(src_hbm.at[pl.ds(wid * EPT, EPT)], src_v)
        plsc.subcore_barrier()

        def issue_chunk(cc):
            bn = lax.rem(cc, RING)
            pltpu.async_copy(dst_hbm.at[pl.ds(wid * EPT + cc * CH, CH)],
                             dst_v.at[bn], dsem.at[bn])
            for hh in range(HALVES):
                pltpu.async_copy(
                    h_hbm.at[src_v.at[pl.ds(cc * CH + hh * HCH, HCH)]],
                    rows_v.at[bn, pl.ds(hh * HCH, HCH)],
                    gsem.at[HALVES * bn + hh])

        def wait_chunk(cc):
            bn = lax.rem(cc, RING)
            pltpu.make_async_copy(dst_hbm.at[pl.ds(wid * EPT + cc * CH, CH)],
                                  dst_v.at[bn], dsem.at[bn]).wait()
            for hh in range(HALVES):
                pltpu.make_async_copy(
                    h_hbm.at[src_v.at[pl.ds(cc * CH + hh * HCH, HCH)]],
                    rows_v.at[bn, pl.ds(hh * HCH, HCH)],
                    gsem.at[HALVES * bn + hh]).wait()

        def drain_scatter(cc):
            bp = lax.rem(cc, RING)
            pltpu.make_async_copy(rows_v.at[bp], acc_sh.at[dst_v.at[bp]],
                                  ssem.at[bp]).wait()

        for c0 in range(LOOK):
            issue_chunk(c0)

        def body(c, carry):
            bn = lax.rem(c, RING)

            @pl.when(c >= RING - LOOK)
            def _():
                drain_scatter(c - (RING - LOOK))

            @pl.when(c + LOOK < NCHUNK)
            def _():
                issue_chunk(c + LOOK)

            wait_chunk(c)
            pltpu.async_copy(rows_v.at[bn], acc_sh.at[dst_v.at[bn]],
                             ssem.at[bn], add=True)
            return carry

        lax.fori_loop(0, NCHUNK, body, 0, unroll=False)
        for cc in range(NCHUNK - (RING - LOOK), NCHUNK):
            bp = cc % RING
            pltpu.make_async_copy(rows_v.at[bp], acc_sh.at[dst_v.at[bp]],
                                  ssem.at[bp]).wait()

        plsc.subcore_barrier()
        pltpu.sync_copy(acc_sh.at[pl.ds(r0, R8)],
                        out_hbm.at[cid, pl.ds(r0, R8)])

        @pl.when(last)
        def _():
            pltpu.sync_copy(acc_sh.at[pl.ds(TAIL_OFF, TAIL)],
                            out_hbm.at[cid, pl.ds(TAIL_OFF, TAIL)])

    return agg_kernel(h, src1, dst1)


BLK = 2000


def _mlp_body(p_ref, w1_ref, b1_ref, w2_ref, b2_ref, o_ref):
    z = p_ref[0] + p_ref[1]
    z = jnp.dot(z, w1_ref[...], preferred_element_type=jnp.float32) + b1_ref[...]
    z = jnp.maximum(z, 0.0)
    z = jnp.dot(z, w2_ref[...], preferred_element_type=jnp.float32) + b2_ref[...]
    o_ref[...] = z


def _mlp(p, W1, b1, W2, b2):
    return pl.pallas_call(
        _mlp_body,
        grid=(N // BLK,),
        in_specs=[
            pl.BlockSpec((NC, BLK, D), lambda i: (0, i, 0)),
            pl.BlockSpec((D, D), lambda i: (0, 0)),
            pl.BlockSpec((1, D), lambda i: (0, 0)),
            pl.BlockSpec((D, D), lambda i: (0, 0)),
            pl.BlockSpec((1, D), lambda i: (0, 0)),
        ],
        out_specs=pl.BlockSpec((BLK, D), lambda i: (i, 0)),
        out_shape=jax.ShapeDtypeStruct((N, D), jnp.float32),
    )(p, W1, b1, W2, b2)


def kernel(x, edge_index, W1_0, b1_0, W2_0, b2_0, W1_1, b1_1, W2_1, b2_1):
    src1 = edge_index[0]
    dst1 = edge_index[1]
    b1_0r = b1_0.reshape(1, D)
    b2_0r = b2_0.reshape(1, D)
    b1_1r = b1_1.reshape(1, D)
    b2_1r = b2_1.reshape(1, D)

    p = _sc_aggregate(x, src1, dst1)
    h = _mlp(p, W1_0, b1_0r, W2_0, b2_0r)
    p = _sc_aggregate(h, src1, dst1)
    return _mlp(p, W1_1, b1_1r, W2_1, b2_1r)
